# baseline (speedup 1.0000x reference)
import functools

import jax
import jax.numpy as jnp
from jax import lax
from jax.experimental import pallas as pl
from jax.experimental.pallas import tpu as pltpu
from jax.experimental.pallas import tpu_sc as plsc

N = 10000
NP = 10240
DX = 128
HJ = 2048
ALPHA = 0.05
C1 = ALPHA * (1.0 - ALPHA)
TOPK = 128

E = 320000
CHUNK = 128
NTILES = 32
NCHUNK = 80
EPT = NCHUNK * CHUNK
E3 = NTILES * EPT
STRIPE = NP // 16

_SC_MESH = plsc.VectorSubcoreMesh(core_axis_name="c", subcore_axis_name="s")


BI2 = 1000
BJ2 = 640


def _k2_body(x_ref, gx_ref, wa_ref, wb_ref, b1_ref, w2_ref, b2_ref,
             zz_ref, dinv_ref, h_s):
    j = pl.program_id(1)
    nj = pl.num_programs(1)

    @pl.when(j == 0)
    def _():
        h = jnp.dot(x_ref[...], wa_ref[...], preferred_element_type=jnp.float32)
        h = h + jnp.dot(gx_ref[...], wb_ref[...],
                        preferred_element_type=jnp.float32)
        h_s[...] = jnp.maximum(h + b1_ref[...], 0.0)

    zz = jnp.dot(h_s[...], w2_ref[...], preferred_element_type=jnp.float32)
    zz = jnp.maximum(zz + b2_ref[...], 0.0)
    zz_ref[...] = zz
    rs = jnp.sum(zz, axis=1, keepdims=True)

    @pl.when(j == 0)
    def _():
        dinv_ref[...] = rs + 1.0

    @pl.when(j > 0)
    def _():
        dinv_ref[...] = dinv_ref[...] + rs

    @pl.when(j == nj - 1)
    def _():
        dinv_ref[...] = lax.rsqrt(dinv_ref[...])


def _run_k2(x, gx, wa, wb, b1, w2p, b2p):
    return pl.pallas_call(
        _k2_body,
        grid=(N // BI2, NP // BJ2),
        in_specs=[
            pl.BlockSpec((BI2, DX), lambda i, j: (i, jnp.int32(0))),
            pl.BlockSpec((BI2, DX), lambda i, j: (i, jnp.int32(0))),
            pl.BlockSpec((DX, HJ), lambda i, j: (jnp.int32(0), jnp.int32(0))),
            pl.BlockSpec((DX, HJ), lambda i, j: (jnp.int32(0), jnp.int32(0))),
            pl.BlockSpec((1, HJ), lambda i, j: (jnp.int32(0), jnp.int32(0))),
            pl.BlockSpec((HJ, BJ2), lambda i, j: (jnp.int32(0), j)),
            pl.BlockSpec((1, BJ2), lambda i, j: (jnp.int32(0), j)),
        ],
        out_specs=[
            pl.BlockSpec((BI2, BJ2), lambda i, j: (i, j)),
            pl.BlockSpec((BI2, 1), lambda i, j: (i, jnp.int32(0))),
        ],
        out_shape=[
            jax.ShapeDtypeStruct((N, NP), jnp.float32),
            jax.ShapeDtypeStruct((N, 1), jnp.float32),
        ],
        scratch_shapes=[pltpu.VMEM((BI2, HJ), jnp.float32)],
    )(x, gx, wa, wb, b1, w2p, b2p)


BC3 = 128


def _k3_body(zz_ref, dc_ref, dr_ref, xp_ref, xn_ref, s_s):
    j = pl.program_id(0)

    dc = dc_ref[...]
    s = (C1 * dc) * zz_ref[...] * dr_ref[...]
    rows = lax.broadcasted_iota(jnp.int32, (N, BC3), 0)
    cols = (j * jnp.int32(BC3)) + lax.broadcasted_iota(jnp.int32, (N, BC3), 1)
    s = jnp.where(rows == cols, s + (C1 * dc * dc + ALPHA), s)
    s_s[...] = s

    lo0 = jnp.zeros((1, BC3), jnp.int32)
    hi0 = jnp.full((1, BC3), 0x7F7FFFFF, jnp.int32)

    def step(_, carry):
        lo, hi = carry
        span = hi - lo
        third = (span + 2) // 3
        m1 = lo + third
        m2 = jnp.minimum(m1 + third, hi)
        t1 = lax.bitcast_convert_type(m1, jnp.float32)
        t2 = lax.bitcast_convert_type(m2, jnp.float32)
        sv = s_s[...]
        c1 = jnp.sum((sv >= t1).astype(jnp.float32), axis=0, keepdims=True)
        c2 = jnp.sum((sv >= t2).astype(jnp.float32), axis=0, keepdims=True)
        p2 = c2 >= float(TOPK)
        p1 = c1 >= float(TOPK)
        lo = jnp.where(p2, m2, jnp.where(p1, m1, lo))
        hi = jnp.where(p2, hi, jnp.where(p1, m2 - 1, m1 - 1))
        return lo, hi

    lo, _ = lax.fori_loop(jnp.int32(0), jnp.int32(20), step, (lo0, hi0))
    kth = lax.bitcast_convert_type(lo, jnp.float32)

    sm = jnp.where(s_s[...] >= kth, s_s[...], 0.0)
    col = jnp.sum(sm, axis=0, keepdims=True)
    sm = sm / jnp.where(col == 0.0, 1.0, col)
    contrib = jnp.dot(sm, xp_ref[...], preferred_element_type=jnp.float32)

    @pl.when(j == 0)
    def _():
        xn_ref[...] = contrib

    @pl.when(j > 0)
    def _():
        xn_ref[...] = xn_ref[...] + contrib


def _run_k3(zz, dinv, dinv_row, x_pad):
    return pl.pallas_call(
        _k3_body,
        grid=(NP // BC3,),
        in_specs=[
            pl.BlockSpec((N, BC3), lambda j: (jnp.int32(0), j)),
            pl.BlockSpec((N, 1), lambda j: (jnp.int32(0), jnp.int32(0))),
            pl.BlockSpec((1, BC3), lambda j: (jnp.int32(0), j)),
            pl.BlockSpec((BC3, DX), lambda j: (j, jnp.int32(0))),
        ],
        out_specs=pl.BlockSpec((N, DX), lambda j: (jnp.int32(0), jnp.int32(0))),
        out_shape=jax.ShapeDtypeStruct((N, DX), jnp.float32),
        scratch_shapes=[pltpu.VMEM((N, BC3), jnp.float32)],
    )(zz, dinv, dinv_row, x_pad)


def _sc_deg_body(dst_hbm, zeros_hbm, out_hbm, dstv, onesv, deg_sh):
    cid = lax.axis_index("c")
    sid = lax.axis_index("s")
    soff = sid * jnp.int32(STRIPE)

    @pl.when(cid == 0)
    def _():
        pltpu.sync_copy(zeros_hbm.at[pl.ds(soff, STRIPE)],
                        deg_sh.at[pl.ds(soff, STRIPE)])
        for jj in range(CHUNK // 16):
            onesv[pl.ds(jj * 16, 16)] = jnp.ones((16,), jnp.float32)
        plsc.subcore_barrier()

        for seg in range(2):
            def body(k, carry, seg=seg):
                base = ((sid + jnp.int32(16 * seg)) * jnp.int32(EPT)
                        + k * jnp.int32(CHUNK))
                pltpu.sync_copy(dst_hbm.at[pl.ds(base, CHUNK)], dstv)
                pltpu.sync_copy(onesv, deg_sh.at[dstv], add=True)
                return carry

            lax.fori_loop(jnp.int32(0), jnp.int32(NCHUNK), body, None)

        plsc.subcore_barrier()
        pltpu.sync_copy(deg_sh.at[pl.ds(soff, STRIPE)],
                        out_hbm.at[pl.ds(soff, STRIPE)])


def _run_sc_deg(dst_pad, zeros1d):
    f = functools.partial(
        pl.kernel,
        out_type=jax.ShapeDtypeStruct((NP,), jnp.float32),
        mesh=_SC_MESH,
        scratch_types=[
            pltpu.VMEM((CHUNK,), jnp.int32),
            pltpu.VMEM((CHUNK,), jnp.float32),
            pltpu.VMEM_SHARED((NP,), jnp.float32),
        ],
    )(_sc_deg_body)
    return f(dst_pad, zeros1d)


def _sc_agg_body(y_hbm, src_hbm, dst_hbm, zeros_hbm, out_hbm,
                 srcv, dstv, rows_v, acc_sh, sem):
    cid = lax.axis_index("c")
    sid = lax.axis_index("s")
    wid = cid * jnp.int32(16) + sid
    soff = sid * jnp.int32(STRIPE)

    pltpu.sync_copy(zeros_hbm.at[pl.ds(soff, STRIPE)],
                    acc_sh.at[pl.ds(soff, STRIPE)])
    plsc.subcore_barrier()

    def body(k, carry):
        base = wid * jnp.int32(EPT) + k * jnp.int32(CHUNK)
        pltpu.sync_copy(src_hbm.at[pl.ds(base, CHUNK)], srcv)
        pltpu.sync_copy(dst_hbm.at[pl.ds(base, CHUNK)], dstv)
        pltpu.async_copy(y_hbm.at[srcv], rows_v, sem).wait()
        pltpu.sync_copy(rows_v, acc_sh.at[dstv], add=True)
        return carry

    lax.fori_loop(jnp.int32(0), jnp.int32(NCHUNK), body, None)

    plsc.subcore_barrier()
    pltpu.sync_copy(acc_sh.at[pl.ds(soff, STRIPE)],
                    out_hbm.at[cid, pl.ds(soff, STRIPE)])


def _run_sc_agg(y, src_pad, dst_pad, zeros2d):
    f = functools.partial(
        pl.kernel,
        out_type=jax.ShapeDtypeStruct((2, NP, DX), jnp.float32),
        mesh=_SC_MESH,
        scratch_types=[
            pltpu.VMEM((CHUNK,), jnp.int32),
            pltpu.VMEM((CHUNK,), jnp.int32),
            pltpu.VMEM((CHUNK, DX), jnp.float32),
            pltpu.VMEM_SHARED((NP, DX), jnp.float32),
            pltpu.SemaphoreType.DMA,
        ],
    )(_sc_agg_body)
    return f(y, src_pad, dst_pad, zeros2d)


BR = 2000


def _k4_body(xn_ref, gx_ref, deg_ref, wea_ref, web_ref, be1_ref, wg1_ref,
             z_ref, dinv_ref, xw1_ref, y1_ref):
    z = jnp.dot(xn_ref[...], wea_ref[...], preferred_element_type=jnp.float32)
    z = z + jnp.dot(gx_ref[...], web_ref[...],
                    preferred_element_type=jnp.float32)
    z = jnp.maximum(z + be1_ref[...], 0.0)
    z_ref[...] = z

    deg = deg_ref[...] + 1.0
    dinv = lax.rsqrt(jnp.maximum(deg, 1.0))
    dinv_ref[...] = dinv

    xw1 = jnp.dot(z + gx_ref[...], wg1_ref[...],
                  preferred_element_type=jnp.float32)
    xw1_ref[...] = xw1
    y1_ref[...] = dinv * xw1


def _run_k4(xn, gx, deg2d, wea, web, be1, wg1):
    return pl.pallas_call(
        _k4_body,
        grid=(N // BR,),
        in_specs=[
            pl.BlockSpec((BR, DX), lambda r: (r, jnp.int32(0))),
            pl.BlockSpec((BR, DX), lambda r: (r, jnp.int32(0))),
            pl.BlockSpec((BR, 1), lambda r: (r, jnp.int32(0))),
            pl.BlockSpec((DX, DX), lambda r: (jnp.int32(0), jnp.int32(0))),
            pl.BlockSpec((DX, DX), lambda r: (jnp.int32(0), jnp.int32(0))),
            pl.BlockSpec((1, DX), lambda r: (jnp.int32(0), jnp.int32(0))),
            pl.BlockSpec((DX, DX), lambda r: (jnp.int32(0), jnp.int32(0))),
        ],
        out_specs=[
            pl.BlockSpec((BR, DX), lambda r: (r, jnp.int32(0))),
            pl.BlockSpec((BR, 1), lambda r: (r, jnp.int32(0))),
            pl.BlockSpec((BR, DX), lambda r: (r, jnp.int32(0))),
            pl.BlockSpec((BR, DX), lambda r: (r, jnp.int32(0))),
        ],
        out_shape=[
            jax.ShapeDtypeStruct((N, DX), jnp.float32),
            jax.ShapeDtypeStruct((N, 1), jnp.float32),
            jax.ShapeDtypeStruct((N, DX), jnp.float32),
            jax.ShapeDtypeStruct((N, DX), jnp.float32),
        ],
    )(xn, gx, deg2d, wea, web, be1, wg1)


def _k5_body(agg_ref, xw1_ref, dinv_ref, bg1_ref, wg2_ref,
             z1_ref, xw2_ref, y2_ref):
    agg = agg_ref[0] + agg_ref[1]
    dinv = dinv_ref[...]
    z1 = jnp.maximum(dinv * agg + dinv * dinv * xw1_ref[...] + bg1_ref[...],
                     0.0)
    z1_ref[...] = z1
    xw2 = jnp.dot(z1, wg2_ref[...], preferred_element_type=jnp.float32)
    xw2_ref[...] = xw2
    y2_ref[...] = dinv * xw2


def _run_k5(agg1, xw1, dinv, bg1, wg2):
    return pl.pallas_call(
        _k5_body,
        grid=(N // BR,),
        in_specs=[
            pl.BlockSpec((2, BR, DX), lambda r: (jnp.int32(0), r, jnp.int32(0))),
            pl.BlockSpec((BR, DX), lambda r: (r, jnp.int32(0))),
            pl.BlockSpec((BR, 1), lambda r: (r, jnp.int32(0))),
            pl.BlockSpec((1, DX), lambda r: (jnp.int32(0), jnp.int32(0))),
            pl.BlockSpec((DX, DX), lambda r: (jnp.int32(0), jnp.int32(0))),
        ],
        out_specs=[
            pl.BlockSpec((BR, DX), lambda r: (r, jnp.int32(0))),
            pl.BlockSpec((BR, DX), lambda r: (r, jnp.int32(0))),
            pl.BlockSpec((BR, DX), lambda r: (r, jnp.int32(0))),
        ],
        out_shape=[
            jax.ShapeDtypeStruct((N, DX), jnp.float32),
            jax.ShapeDtypeStruct((N, DX), jnp.float32),
            jax.ShapeDtypeStruct((N, DX), jnp.float32),
        ],
    )(agg1, xw1, dinv, bg1, wg2)


def _k6_body(agg_ref, xw2_ref, dinv_ref, z_ref, z1_ref, bg2_ref,
             wa_ref, wb_ref, wc_ref, be2_ref, we3_ref, be3_ref,
             wo_ref, bo_ref, out_ref):
    agg = agg_ref[0] + agg_ref[1]
    dinv = dinv_ref[...]
    z2 = jnp.maximum(dinv * agg + dinv * dinv * xw2_ref[...] + bg2_ref[...],
                     0.0)
    t = jnp.dot(z_ref[...], wa_ref[...], preferred_element_type=jnp.float32)
    t = t + jnp.dot(z1_ref[...], wb_ref[...],
                    preferred_element_type=jnp.float32)
    t = t + jnp.dot(z2, wc_ref[...], preferred_element_type=jnp.float32)
    t = jnp.maximum(t + be2_ref[...], 0.0)
    t = jnp.maximum(jnp.dot(t, we3_ref[...],
                            preferred_element_type=jnp.float32) + be3_ref[...],
                    0.0)
    out_ref[...] = jnp.dot(t, wo_ref[...],
                           preferred_element_type=jnp.float32) + bo_ref[...]


def _run_k6(agg2, xw2, dinv, z, z1, bg2, wa, wb, wc, be2, we3, be3, wo, bo):
    nl = wo.shape[1]
    return pl.pallas_call(
        _k6_body,
        grid=(N // BR,),
        in_specs=[
            pl.BlockSpec((2, BR, DX), lambda r: (jnp.int32(0), r, jnp.int32(0))),
            pl.BlockSpec((BR, DX), lambda r: (r, jnp.int32(0))),
            pl.BlockSpec((BR, 1), lambda r: (r, jnp.int32(0))),
            pl.BlockSpec((BR, DX), lambda r: (r, jnp.int32(0))),
            pl.BlockSpec((BR, DX), lambda r: (r, jnp.int32(0))),
            pl.BlockSpec((1, DX), lambda r: (jnp.int32(0), jnp.int32(0))),
            pl.BlockSpec((DX, DX), lambda r: (jnp.int32(0), jnp.int32(0))),
            pl.BlockSpec((DX, DX), lambda r: (jnp.int32(0), jnp.int32(0))),
            pl.BlockSpec((DX, DX), lambda r: (jnp.int32(0), jnp.int32(0))),
            pl.BlockSpec((1, DX), lambda r: (jnp.int32(0), jnp.int32(0))),
            pl.BlockSpec((DX, DX), lambda r: (jnp.int32(0), jnp.int32(0))),
            pl.BlockSpec((1, DX), lambda r: (jnp.int32(0), jnp.int32(0))),
            pl.BlockSpec((DX, nl), lambda r: (jnp.int32(0), jnp.int32(0))),
            pl.BlockSpec((1, nl), lambda r: (jnp.int32(0), jnp.int32(0))),
        ],
        out_specs=pl.BlockSpec((BR, nl), lambda r: (r, jnp.int32(0))),
        out_shape=jax.ShapeDtypeStruct((N, nl), jnp.float32),
    )(agg2, xw2, dinv, z, z1, bg2, wa, wb, wc, be2, we3, be3, wo, bo)


def kernel(x, gx, edge_index, W_e1, b_e1, W_dr, b_dr, W_g1, b_g1, W_g2, b_g2,
           W_e2, b_e2, W_e3, b_e3, W_j1, b_j1, W_j2, b_j2, W_out, b_out):
    f32 = jnp.float32
    (x, gx, W_e1, b_e1, W_g1, b_g1, W_g2, b_g2, W_e2, b_e2, W_e3, b_e3,
     W_j1, b_j1, W_j2, b_j2, W_out, b_out) = (
        a.astype(f32) for a in
        (x, gx, W_e1, b_e1, W_g1, b_g1, W_g2, b_g2, W_e2, b_e2, W_e3, b_e3,
         W_j1, b_j1, W_j2, b_j2, W_out, b_out))

    w_j1a, w_j1b = W_j1[:DX], W_j1[DX:]
    b_j1r = b_j1.reshape(1, HJ)
    w_j2p = jnp.pad(W_j2, ((0, 0), (0, NP - N)))
    b_j2p = jnp.pad(b_j2, (0, NP - N)).reshape(1, NP)
    x_pad = jnp.pad(x, ((0, NP - N), (0, 0)))

    src = edge_index[0].astype(jnp.int32)
    dst = edge_index[1].astype(jnp.int32)
    src_pad = jnp.pad(src, (0, E3 - E))
    dst_pad = jnp.pad(dst, (0, E3 - E), constant_values=N)
    zeros1d = jnp.zeros((NP,), f32)
    zeros2d = jnp.zeros((NP, DX), f32)

    w_e1a, w_e1b = W_e1[:DX], W_e1[DX:]
    w_e2a, w_e2b, w_e2c = W_e2[:DX], W_e2[DX:2 * DX], W_e2[2 * DX:]

    deg1d = _run_sc_deg(dst_pad, zeros1d)
    deg2d = deg1d.reshape(NP, 1)

    zz, dinv = _run_k2(x, gx, w_j1a, w_j1b, b_j1r, w_j2p, b_j2p)
    dinv_row = jnp.concatenate(
        [dinv.reshape(1, N), jnp.ones((1, NP - N), f32)], axis=1)
    xn = _run_k3(zz, dinv, dinv_row, x_pad)

    z, gdinv, xw1, y1 = _run_k4(xn, gx, deg2d, w_e1a, w_e1b,
                                b_e1.reshape(1, DX), W_g1)
    agg1 = _run_sc_agg(y1, src_pad, dst_pad, zeros2d)
    z1, xw2, y2 = _run_k5(agg1, xw1, gdinv, b_g1.reshape(1, DX), W_g2)
    agg2 = _run_sc_agg(y2, src_pad, dst_pad, zeros2d)
    out = _run_k6(agg2, xw2, gdinv, z, z1, b_g2.reshape(1, DX),
                  w_e2a, w_e2b, w_e2c, b_e2.reshape(1, DX),
                  W_e3, b_e3.reshape(1, DX), W_out, b_out.reshape(1, -1))
    return out.astype(jnp.float64)


# speedup vs baseline: 1.0303x; 1.0303x over previous
import functools

import jax
import jax.numpy as jnp
from jax import lax
from jax.experimental import pallas as pl
from jax.experimental.pallas import tpu as pltpu
from jax.experimental.pallas import tpu_sc as plsc

N = 10000
NP = 10240
DX = 128
HJ = 2048
ALPHA = 0.05
C1 = ALPHA * (1.0 - ALPHA)
TOPK = 128

E = 320000
CHUNK = 128
NTILES = 32
NCHUNK = 80
EPT = NCHUNK * CHUNK
ESTRIDE = EPT + CHUNK
E3 = NTILES * ESTRIDE
STRIPE = NP // 16

_SC_MESH = plsc.VectorSubcoreMesh(core_axis_name="c", subcore_axis_name="s")


BI2 = 1000
BJ2 = 640


def _k2_body(x_ref, gx_ref, wa_ref, wb_ref, b1_ref, w2_ref, b2_ref,
             zz_ref, dinv_ref, h_s):
    j = pl.program_id(1)
    nj = pl.num_programs(1)

    @pl.when(j == 0)
    def _():
        h = jnp.dot(x_ref[...], wa_ref[...], preferred_element_type=jnp.float32)
        h = h + jnp.dot(gx_ref[...], wb_ref[...],
                        preferred_element_type=jnp.float32)
        h_s[...] = jnp.maximum(h + b1_ref[...], 0.0)

    zz = jnp.dot(h_s[...], w2_ref[...], preferred_element_type=jnp.float32)
    zz = jnp.maximum(zz + b2_ref[...], 0.0)
    zz_ref[...] = zz
    rs = jnp.sum(zz, axis=1, keepdims=True)

    @pl.when(j == 0)
    def _():
        dinv_ref[...] = rs + 1.0

    @pl.when(j > 0)
    def _():
        dinv_ref[...] = dinv_ref[...] + rs

    @pl.when(j == nj - 1)
    def _():
        dinv_ref[...] = lax.rsqrt(dinv_ref[...])


def _run_k2(x, gx, wa, wb, b1, w2p, b2p):
    return pl.pallas_call(
        _k2_body,
        grid=(N // BI2, NP // BJ2),
        in_specs=[
            pl.BlockSpec((BI2, DX), lambda i, j: (i, jnp.int32(0))),
            pl.BlockSpec((BI2, DX), lambda i, j: (i, jnp.int32(0))),
            pl.BlockSpec((DX, HJ), lambda i, j: (jnp.int32(0), jnp.int32(0))),
            pl.BlockSpec((DX, HJ), lambda i, j: (jnp.int32(0), jnp.int32(0))),
            pl.BlockSpec((1, HJ), lambda i, j: (jnp.int32(0), jnp.int32(0))),
            pl.BlockSpec((HJ, BJ2), lambda i, j: (jnp.int32(0), j)),
            pl.BlockSpec((1, BJ2), lambda i, j: (jnp.int32(0), j)),
        ],
        out_specs=[
            pl.BlockSpec((BI2, BJ2), lambda i, j: (i, j)),
            pl.BlockSpec((BI2, 1), lambda i, j: (i, jnp.int32(0))),
        ],
        out_shape=[
            jax.ShapeDtypeStruct((N, NP), jnp.float32),
            jax.ShapeDtypeStruct((N, 1), jnp.float32),
        ],
        scratch_shapes=[pltpu.VMEM((BI2, HJ), jnp.float32)],
    )(x, gx, wa, wb, b1, w2p, b2p)


BC3 = 128


def _k3_body(zz_ref, dc_ref, dr_ref, xp_ref, xn_ref, s_s):
    j = pl.program_id(0)

    dc = dc_ref[...]
    s = (C1 * dc) * zz_ref[...] * dr_ref[...]
    rows = lax.broadcasted_iota(jnp.int32, (N, BC3), 0)
    cols = (j * jnp.int32(BC3)) + lax.broadcasted_iota(jnp.int32, (N, BC3), 1)
    s = jnp.where(rows == cols, s + (C1 * dc * dc + ALPHA), s)
    s_s[...] = s

    lo0 = jnp.zeros((1, BC3), jnp.int32)
    hi0 = jnp.full((1, BC3), 0x7F7FFFFF, jnp.int32)

    def step(_, carry):
        lo, hi = carry
        span = hi - lo
        third = (span + 2) // 3
        m1 = lo + third
        m2 = jnp.minimum(m1 + third, hi)
        t1 = lax.bitcast_convert_type(m1, jnp.float32)
        t2 = lax.bitcast_convert_type(m2, jnp.float32)
        sv = s_s[...]
        c1 = jnp.sum((sv >= t1).astype(jnp.float32), axis=0, keepdims=True)
        c2 = jnp.sum((sv >= t2).astype(jnp.float32), axis=0, keepdims=True)
        p2 = c2 >= float(TOPK)
        p1 = c1 >= float(TOPK)
        lo = jnp.where(p2, m2, jnp.where(p1, m1, lo))
        hi = jnp.where(p2, hi, jnp.where(p1, m2 - 1, m1 - 1))
        return lo, hi

    lo, _ = lax.fori_loop(jnp.int32(0), jnp.int32(20), step, (lo0, hi0))
    kth = lax.bitcast_convert_type(lo, jnp.float32)

    sm = jnp.where(s_s[...] >= kth, s_s[...], 0.0)
    col = jnp.sum(sm, axis=0, keepdims=True)
    sm = sm / jnp.where(col == 0.0, 1.0, col)
    contrib = jnp.dot(sm, xp_ref[...], preferred_element_type=jnp.float32)

    @pl.when(j == 0)
    def _():
        xn_ref[...] = contrib

    @pl.when(j > 0)
    def _():
        xn_ref[...] = xn_ref[...] + contrib


def _run_k3(zz, dinv, dinv_row, x_pad):
    return pl.pallas_call(
        _k3_body,
        grid=(NP // BC3,),
        in_specs=[
            pl.BlockSpec((N, BC3), lambda j: (jnp.int32(0), j)),
            pl.BlockSpec((N, 1), lambda j: (jnp.int32(0), jnp.int32(0))),
            pl.BlockSpec((1, BC3), lambda j: (jnp.int32(0), j)),
            pl.BlockSpec((BC3, DX), lambda j: (j, jnp.int32(0))),
        ],
        out_specs=pl.BlockSpec((N, DX), lambda j: (jnp.int32(0), jnp.int32(0))),
        out_shape=jax.ShapeDtypeStruct((N, DX), jnp.float32),
        scratch_shapes=[pltpu.VMEM((N, BC3), jnp.float32)],
    )(zz, dinv, dinv_row, x_pad)


def _sc_deg_body(dst_hbm, zeros_hbm, out_hbm, dstv, onesv, deg_sh):
    cid = lax.axis_index("c")
    sid = lax.axis_index("s")
    soff = sid * jnp.int32(STRIPE)

    @pl.when(cid == 0)
    def _():
        pltpu.sync_copy(zeros_hbm.at[pl.ds(soff, STRIPE)],
                        deg_sh.at[pl.ds(soff, STRIPE)])
        for jj in range(CHUNK // 16):
            onesv[pl.ds(jj * 16, 16)] = jnp.ones((16,), jnp.float32)
        plsc.subcore_barrier()

        for seg in range(2):
            def body(k, carry, seg=seg):
                base = ((sid + jnp.int32(16 * seg)) * jnp.int32(ESTRIDE)
                        + k * jnp.int32(CHUNK))
                pltpu.sync_copy(dst_hbm.at[pl.ds(base, CHUNK)], dstv)
                pltpu.sync_copy(onesv, deg_sh.at[dstv], add=True)
                return carry

            lax.fori_loop(jnp.int32(0), jnp.int32(NCHUNK), body, None)

        plsc.subcore_barrier()
        pltpu.sync_copy(deg_sh.at[pl.ds(soff, STRIPE)],
                        out_hbm.at[pl.ds(soff, STRIPE)])


def _run_sc_deg(dst_pad, zeros1d):
    f = functools.partial(
        pl.kernel,
        out_type=jax.ShapeDtypeStruct((NP,), jnp.float32),
        mesh=_SC_MESH,
        scratch_types=[
            pltpu.VMEM((CHUNK,), jnp.int32),
            pltpu.VMEM((CHUNK,), jnp.float32),
            pltpu.VMEM_SHARED((NP,), jnp.float32),
        ],
    )(_sc_deg_body)
    return f(dst_pad, zeros1d)


def _sc_agg_body(y_hbm, src_hbm, dst_hbm, zeros_hbm, out_hbm,
                 srcv0, srcv1, dstv0, dstv1, rows0, rows1, acc_sh,
                 semi0, semi1, semg0, semg1):
    cid = lax.axis_index("c")
    sid = lax.axis_index("s")
    wid = cid * jnp.int32(16) + sid
    soff = sid * jnp.int32(STRIPE)

    pltpu.sync_copy(zeros_hbm.at[pl.ds(soff, STRIPE)],
                    acc_sh.at[pl.ds(soff, STRIPE)])
    plsc.subcore_barrier()

    srcv = (srcv0, srcv1)
    dstv = (dstv0, dstv1)
    rows = (rows0, rows1)
    semi = (semi0, semi1)
    semg = (semg0, semg1)
    nc = jnp.int32(NCHUNK)
    ebase = wid * jnp.int32(ESTRIDE)

    def idx_start(k, b):
        base = ebase + k * jnp.int32(CHUNK)
        pltpu.async_copy(src_hbm.at[pl.ds(base, CHUNK)], srcv[b], semi[b])
        pltpu.async_copy(dst_hbm.at[pl.ds(base, CHUNK)], dstv[b], semi[b])

    def idx_wait(k, b):
        base = ebase + k * jnp.int32(CHUNK)
        pltpu.make_async_copy(src_hbm.at[pl.ds(base, CHUNK)], srcv[b],
                              semi[b]).wait()
        pltpu.make_async_copy(dst_hbm.at[pl.ds(base, CHUNK)], dstv[b],
                              semi[b]).wait()

    idx_start(jnp.int32(0), 0)
    idx_start(jnp.int32(1), 1)
    idx_wait(jnp.int32(0), 0)
    pltpu.async_copy(y_hbm.at[srcv[0]], rows[0], semg[0])

    def body(k2, carry):
        for b in range(2):
            k = k2 * jnp.int32(2) + jnp.int32(b)
            nb = 1 - b

            @pl.when(k + 1 < nc)
            def _(k=k, nb=nb):
                idx_wait(k + jnp.int32(1), nb)
                pltpu.async_copy(y_hbm.at[srcv[nb]], rows[nb], semg[nb])

            pltpu.make_async_copy(y_hbm.at[srcv[b]], rows[b],
                                  semg[b]).wait()
            pltpu.sync_copy(rows[b], acc_sh.at[dstv[b]], add=True)

            @pl.when(k + 2 < nc)
            def _(k=k, b=b):
                idx_start(k + jnp.int32(2), b)
        return carry

    lax.fori_loop(jnp.int32(0), jnp.int32(NCHUNK // 2), body, None)

    plsc.subcore_barrier()
    pltpu.sync_copy(acc_sh.at[pl.ds(soff, STRIPE)],
                    out_hbm.at[cid, pl.ds(soff, STRIPE)])


def _run_sc_agg(y, src_pad, dst_pad, zeros2d):
    f = functools.partial(
        pl.kernel,
        out_type=jax.ShapeDtypeStruct((2, NP, DX), jnp.float32),
        mesh=_SC_MESH,
        scratch_types=[
            pltpu.VMEM((CHUNK,), jnp.int32),
            pltpu.VMEM((CHUNK,), jnp.int32),
            pltpu.VMEM((CHUNK,), jnp.int32),
            pltpu.VMEM((CHUNK,), jnp.int32),
            pltpu.VMEM((CHUNK, DX), jnp.float32),
            pltpu.VMEM((CHUNK, DX), jnp.float32),
            pltpu.VMEM_SHARED((NP, DX), jnp.float32),
            pltpu.SemaphoreType.DMA,
            pltpu.SemaphoreType.DMA,
            pltpu.SemaphoreType.DMA,
            pltpu.SemaphoreType.DMA,
        ],
    )(_sc_agg_body)
    return f(y, src_pad, dst_pad, zeros2d)


BR = 2000


def _k4_body(xn_ref, gx_ref, deg_ref, wea_ref, web_ref, be1_ref, wg1_ref,
             z_ref, dinv_ref, xw1_ref, y1_ref):
    z = jnp.dot(xn_ref[...], wea_ref[...], preferred_element_type=jnp.float32)
    z = z + jnp.dot(gx_ref[...], web_ref[...],
                    preferred_element_type=jnp.float32)
    z = jnp.maximum(z + be1_ref[...], 0.0)
    z_ref[...] = z

    deg = deg_ref[...] + 1.0
    dinv = lax.rsqrt(jnp.maximum(deg, 1.0))
    dinv_ref[...] = dinv

    xw1 = jnp.dot(z + gx_ref[...], wg1_ref[...],
                  preferred_element_type=jnp.float32)
    xw1_ref[...] = xw1
    y1_ref[...] = dinv * xw1


def _run_k4(xn, gx, deg2d, wea, web, be1, wg1):
    return pl.pallas_call(
        _k4_body,
        grid=(N // BR,),
        in_specs=[
            pl.BlockSpec((BR, DX), lambda r: (r, jnp.int32(0))),
            pl.BlockSpec((BR, DX), lambda r: (r, jnp.int32(0))),
            pl.BlockSpec((BR, 1), lambda r: (r, jnp.int32(0))),
            pl.BlockSpec((DX, DX), lambda r: (jnp.int32(0), jnp.int32(0))),
            pl.BlockSpec((DX, DX), lambda r: (jnp.int32(0), jnp.int32(0))),
            pl.BlockSpec((1, DX), lambda r: (jnp.int32(0), jnp.int32(0))),
            pl.BlockSpec((DX, DX), lambda r: (jnp.int32(0), jnp.int32(0))),
        ],
        out_specs=[
            pl.BlockSpec((BR, DX), lambda r: (r, jnp.int32(0))),
            pl.BlockSpec((BR, 1), lambda r: (r, jnp.int32(0))),
            pl.BlockSpec((BR, DX), lambda r: (r, jnp.int32(0))),
            pl.BlockSpec((BR, DX), lambda r: (r, jnp.int32(0))),
        ],
        out_shape=[
            jax.ShapeDtypeStruct((N, DX), jnp.float32),
            jax.ShapeDtypeStruct((N, 1), jnp.float32),
            jax.ShapeDtypeStruct((N, DX), jnp.float32),
            jax.ShapeDtypeStruct((N, DX), jnp.float32),
        ],
    )(xn, gx, deg2d, wea, web, be1, wg1)


def _k5_body(agg_ref, xw1_ref, dinv_ref, bg1_ref, wg2_ref,
             z1_ref, xw2_ref, y2_ref):
    agg = agg_ref[0] + agg_ref[1]
    dinv = dinv_ref[...]
    z1 = jnp.maximum(dinv * agg + dinv * dinv * xw1_ref[...] + bg1_ref[...],
                     0.0)
    z1_ref[...] = z1
    xw2 = jnp.dot(z1, wg2_ref[...], preferred_element_type=jnp.float32)
    xw2_ref[...] = xw2
    y2_ref[...] = dinv * xw2


def _run_k5(agg1, xw1, dinv, bg1, wg2):
    return pl.pallas_call(
        _k5_body,
        grid=(N // BR,),
        in_specs=[
            pl.BlockSpec((2, BR, DX), lambda r: (jnp.int32(0), r, jnp.int32(0))),
            pl.BlockSpec((BR, DX), lambda r: (r, jnp.int32(0))),
            pl.BlockSpec((BR, 1), lambda r: (r, jnp.int32(0))),
            pl.BlockSpec((1, DX), lambda r: (jnp.int32(0), jnp.int32(0))),
            pl.BlockSpec((DX, DX), lambda r: (jnp.int32(0), jnp.int32(0))),
        ],
        out_specs=[
            pl.BlockSpec((BR, DX), lambda r: (r, jnp.int32(0))),
            pl.BlockSpec((BR, DX), lambda r: (r, jnp.int32(0))),
            pl.BlockSpec((BR, DX), lambda r: (r, jnp.int32(0))),
        ],
        out_shape=[
            jax.ShapeDtypeStruct((N, DX), jnp.float32),
            jax.ShapeDtypeStruct((N, DX), jnp.float32),
            jax.ShapeDtypeStruct((N, DX), jnp.float32),
        ],
    )(agg1, xw1, dinv, bg1, wg2)


def _k6_body(agg_ref, xw2_ref, dinv_ref, z_ref, z1_ref, bg2_ref,
             wa_ref, wb_ref, wc_ref, be2_ref, we3_ref, be3_ref,
             wo_ref, bo_ref, out_ref):
    agg = agg_ref[0] + agg_ref[1]
    dinv = dinv_ref[...]
    z2 = jnp.maximum(dinv * agg + dinv * dinv * xw2_ref[...] + bg2_ref[...],
                     0.0)
    t = jnp.dot(z_ref[...], wa_ref[...], preferred_element_type=jnp.float32)
    t = t + jnp.dot(z1_ref[...], wb_ref[...],
                    preferred_element_type=jnp.float32)
    t = t + jnp.dot(z2, wc_ref[...], preferred_element_type=jnp.float32)
    t = jnp.maximum(t + be2_ref[...], 0.0)
    t = jnp.maximum(jnp.dot(t, we3_ref[...],
                            preferred_element_type=jnp.float32) + be3_ref[...],
                    0.0)
    out_ref[...] = jnp.dot(t, wo_ref[...],
                           preferred_element_type=jnp.float32) + bo_ref[...]


def _run_k6(agg2, xw2, dinv, z, z1, bg2, wa, wb, wc, be2, we3, be3, wo, bo):
    nl = wo.shape[1]
    return pl.pallas_call(
        _k6_body,
        grid=(N // BR,),
        in_specs=[
            pl.BlockSpec((2, BR, DX), lambda r: (jnp.int32(0), r, jnp.int32(0))),
            pl.BlockSpec((BR, DX), lambda r: (r, jnp.int32(0))),
            pl.BlockSpec((BR, 1), lambda r: (r, jnp.int32(0))),
            pl.BlockSpec((BR, DX), lambda r: (r, jnp.int32(0))),
            pl.BlockSpec((BR, DX), lambda r: (r, jnp.int32(0))),
            pl.BlockSpec((1, DX), lambda r: (jnp.int32(0), jnp.int32(0))),
            pl.BlockSpec((DX, DX), lambda r: (jnp.int32(0), jnp.int32(0))),
            pl.BlockSpec((DX, DX), lambda r: (jnp.int32(0), jnp.int32(0))),
            pl.BlockSpec((DX, DX), lambda r: (jnp.int32(0), jnp.int32(0))),
            pl.BlockSpec((1, DX), lambda r: (jnp.int32(0), jnp.int32(0))),
            pl.BlockSpec((DX, DX), lambda r: (jnp.int32(0), jnp.int32(0))),
            pl.BlockSpec((1, DX), lambda r: (jnp.int32(0), jnp.int32(0))),
            pl.BlockSpec((DX, nl), lambda r: (jnp.int32(0), jnp.int32(0))),
            pl.BlockSpec((1, nl), lambda r: (jnp.int32(0), jnp.int32(0))),
        ],
        out_specs=pl.BlockSpec((BR, nl), lambda r: (r, jnp.int32(0))),
        out_shape=jax.ShapeDtypeStruct((N, nl), jnp.float32),
    )(agg2, xw2, dinv, z, z1, bg2, wa, wb, wc, be2, we3, be3, wo, bo)


def kernel(x, gx, edge_index, W_e1, b_e1, W_dr, b_dr, W_g1, b_g1, W_g2, b_g2,
           W_e2, b_e2, W_e3, b_e3, W_j1, b_j1, W_j2, b_j2, W_out, b_out):
    f32 = jnp.float32
    (x, gx, W_e1, b_e1, W_g1, b_g1, W_g2, b_g2, W_e2, b_e2, W_e3, b_e3,
     W_j1, b_j1, W_j2, b_j2, W_out, b_out) = (
        a.astype(f32) for a in
        (x, gx, W_e1, b_e1, W_g1, b_g1, W_g2, b_g2, W_e2, b_e2, W_e3, b_e3,
         W_j1, b_j1, W_j2, b_j2, W_out, b_out))

    w_j1a, w_j1b = W_j1[:DX], W_j1[DX:]
    b_j1r = b_j1.reshape(1, HJ)
    w_j2p = jnp.pad(W_j2, ((0, 0), (0, NP - N)))
    b_j2p = jnp.pad(b_j2, (0, NP - N)).reshape(1, NP)
    x_pad = jnp.pad(x, ((0, NP - N), (0, 0)))

    src = edge_index[0].astype(jnp.int32)
    dst = edge_index[1].astype(jnp.int32)
    src_pad = jnp.pad(
        jnp.pad(src, (0, NTILES * EPT - E)).reshape(NTILES, EPT),
        ((0, 0), (0, ESTRIDE - EPT))).reshape(E3)
    dst_pad = jnp.pad(
        jnp.pad(dst, (0, NTILES * EPT - E),
                constant_values=N).reshape(NTILES, EPT),
        ((0, 0), (0, ESTRIDE - EPT)), constant_values=N).reshape(E3)
    zeros1d = jnp.zeros((NP,), f32)
    zeros2d = jnp.zeros((NP, DX), f32)

    w_e1a, w_e1b = W_e1[:DX], W_e1[DX:]
    w_e2a, w_e2b, w_e2c = W_e2[:DX], W_e2[DX:2 * DX], W_e2[2 * DX:]

    deg1d = _run_sc_deg(dst_pad, zeros1d)
    deg2d = deg1d.reshape(NP, 1)

    zz, dinv = _run_k2(x, gx, w_j1a, w_j1b, b_j1r, w_j2p, b_j2p)
    dinv_row = jnp.concatenate(
        [dinv.reshape(1, N), jnp.ones((1, NP - N), f32)], axis=1)
    xn = _run_k3(zz, dinv, dinv_row, x_pad)

    z, gdinv, xw1, y1 = _run_k4(xn, gx, deg2d, w_e1a, w_e1b,
                                b_e1.reshape(1, DX), W_g1)
    agg1 = _run_sc_agg(y1, src_pad, dst_pad, zeros2d)
    z1, xw2, y2 = _run_k5(agg1, xw1, gdinv, b_g1.reshape(1, DX), W_g2)
    agg2 = _run_sc_agg(y2, src_pad, dst_pad, zeros2d)
    out = _run_k6(agg2, xw2, gdinv, z, z1, b_g2.reshape(1, DX),
                  w_e2a, w_e2b, w_e2c, b_e2.reshape(1, DX),
                  W_e3, b_e3.reshape(1, DX), W_out, b_out.reshape(1, -1))
    return out.astype(jnp.float64)


# speedup vs baseline: 1.0788x; 1.0471x over previous
import functools

import jax
import jax.numpy as jnp
from jax import lax
from jax.experimental import pallas as pl
from jax.experimental.pallas import tpu as pltpu
from jax.experimental.pallas import tpu_sc as plsc

N = 10000
NP = 10240
DX = 128
HJ = 2048
ALPHA = 0.05
C1 = ALPHA * (1.0 - ALPHA)
TOPK = 128

E = 320000
CHUNK = 128
NTILES = 32
NCHUNK = 79
EPT = NCHUNK * CHUNK
E3 = NTILES * EPT
STRIPE = NP // 16

_SC_MESH = plsc.VectorSubcoreMesh(core_axis_name="c", subcore_axis_name="s")


BI2 = 1000
BJ2 = 640


def _k2_body(x_ref, gx_ref, wa_ref, wb_ref, b1_ref, w2_ref, b2_ref,
             zz_ref, dinv_ref, h_s):
    j = pl.program_id(1)
    nj = pl.num_programs(1)

    @pl.when(j == 0)
    def _():
        h = jnp.dot(x_ref[...], wa_ref[...], preferred_element_type=jnp.float32)
        h = h + jnp.dot(gx_ref[...], wb_ref[...],
                        preferred_element_type=jnp.float32)
        h_s[...] = jnp.maximum(h + b1_ref[...], 0.0)

    zz = jnp.dot(h_s[...], w2_ref[...], preferred_element_type=jnp.float32)
    zz = jnp.maximum(zz + b2_ref[...], 0.0)
    zz_ref[...] = zz
    rs = jnp.sum(zz, axis=1, keepdims=True)

    @pl.when(j == 0)
    def _():
        dinv_ref[...] = rs + 1.0

    @pl.when(j > 0)
    def _():
        dinv_ref[...] = dinv_ref[...] + rs

    @pl.when(j == nj - 1)
    def _():
        dinv_ref[...] = lax.rsqrt(dinv_ref[...])


def _run_k2(x, gx, wa, wb, b1, w2p, b2p):
    return pl.pallas_call(
        _k2_body,
        grid=(N // BI2, NP // BJ2),
        in_specs=[
            pl.BlockSpec((BI2, DX), lambda i, j: (i, jnp.int32(0))),
            pl.BlockSpec((BI2, DX), lambda i, j: (i, jnp.int32(0))),
            pl.BlockSpec((DX, HJ), lambda i, j: (jnp.int32(0), jnp.int32(0))),
            pl.BlockSpec((DX, HJ), lambda i, j: (jnp.int32(0), jnp.int32(0))),
            pl.BlockSpec((1, HJ), lambda i, j: (jnp.int32(0), jnp.int32(0))),
            pl.BlockSpec((HJ, BJ2), lambda i, j: (jnp.int32(0), j)),
            pl.BlockSpec((1, BJ2), lambda i, j: (jnp.int32(0), j)),
        ],
        out_specs=[
            pl.BlockSpec((BI2, BJ2), lambda i, j: (i, j)),
            pl.BlockSpec((BI2, 1), lambda i, j: (i, jnp.int32(0))),
        ],
        out_shape=[
            jax.ShapeDtypeStruct((N, NP), jnp.float32),
            jax.ShapeDtypeStruct((N, 1), jnp.float32),
        ],
        scratch_shapes=[pltpu.VMEM((BI2, HJ), jnp.float32)],
    )(x, gx, wa, wb, b1, w2p, b2p)


BC3 = 128


def _k3_body(zz_ref, dc_ref, dr_ref, xp_ref, xn_ref, s_s):
    j = pl.program_id(0)

    dc = dc_ref[...]
    s = (C1 * dc) * zz_ref[...] * dr_ref[...]
    rows = lax.broadcasted_iota(jnp.int32, (N, BC3), 0)
    cols = (j * jnp.int32(BC3)) + lax.broadcasted_iota(jnp.int32, (N, BC3), 1)
    s = jnp.where(rows == cols, s + (C1 * dc * dc + ALPHA), s)
    s_s[...] = s

    lo0 = jnp.zeros((1, BC3), jnp.int32)
    hi0 = jnp.full((1, BC3), 0x7F7FFFFF, jnp.int32)

    def step(_, carry):
        lo, hi = carry
        span = hi - lo
        third = (span + 2) // 3
        m1 = lo + third
        m2 = jnp.minimum(m1 + third, hi)
        t1 = lax.bitcast_convert_type(m1, jnp.float32)
        t2 = lax.bitcast_convert_type(m2, jnp.float32)
        sv = s_s[...]
        c1 = jnp.sum((sv >= t1).astype(jnp.float32), axis=0, keepdims=True)
        c2 = jnp.sum((sv >= t2).astype(jnp.float32), axis=0, keepdims=True)
        p2 = c2 >= float(TOPK)
        p1 = c1 >= float(TOPK)
        lo = jnp.where(p2, m2, jnp.where(p1, m1, lo))
        hi = jnp.where(p2, hi, jnp.where(p1, m2 - 1, m1 - 1))
        return lo, hi

    lo, _ = lax.fori_loop(jnp.int32(0), jnp.int32(20), step, (lo0, hi0))
    kth = lax.bitcast_convert_type(lo, jnp.float32)

    sm = jnp.where(s_s[...] >= kth, s_s[...], 0.0)
    col = jnp.sum(sm, axis=0, keepdims=True)
    sm = sm / jnp.where(col == 0.0, 1.0, col)
    contrib = jnp.dot(sm, xp_ref[...], preferred_element_type=jnp.float32)

    @pl.when(j == 0)
    def _():
        xn_ref[...] = contrib

    @pl.when(j > 0)
    def _():
        xn_ref[...] = xn_ref[...] + contrib


def _run_k3(zz, dinv, dinv_row, x_pad):
    return pl.pallas_call(
        _k3_body,
        grid=(NP // BC3,),
        in_specs=[
            pl.BlockSpec((N, BC3), lambda j: (jnp.int32(0), j)),
            pl.BlockSpec((N, 1), lambda j: (jnp.int32(0), jnp.int32(0))),
            pl.BlockSpec((1, BC3), lambda j: (jnp.int32(0), j)),
            pl.BlockSpec((BC3, DX), lambda j: (j, jnp.int32(0))),
        ],
        out_specs=pl.BlockSpec((N, DX), lambda j: (jnp.int32(0), jnp.int32(0))),
        out_shape=jax.ShapeDtypeStruct((N, DX), jnp.float32),
        scratch_shapes=[pltpu.VMEM((N, BC3), jnp.float32)],
    )(zz, dinv, dinv_row, x_pad)


def _sc_deg_body(dst_hbm, zeros_hbm, out_hbm, dstv, onesv, deg_sh):
    cid = lax.axis_index("c")
    sid = lax.axis_index("s")
    soff = sid * jnp.int32(STRIPE)

    @pl.when(cid == 0)
    def _():
        pltpu.sync_copy(zeros_hbm.at[pl.ds(soff, STRIPE)],
                        deg_sh.at[pl.ds(soff, STRIPE)])
        for jj in range(CHUNK // 16):
            onesv[pl.ds(jj * 16, 16)] = jnp.ones((16,), jnp.float32)
        plsc.subcore_barrier()

        for seg in range(2):
            def body(k, carry, seg=seg):
                base = ((sid + jnp.int32(16 * seg)) * jnp.int32(EPT)
                        + k * jnp.int32(CHUNK))
                pltpu.sync_copy(dst_hbm.at[pl.ds(base, CHUNK)], dstv)
                pltpu.sync_copy(onesv, deg_sh.at[dstv], add=True)
                return carry

            lax.fori_loop(jnp.int32(0), jnp.int32(NCHUNK), body, None)

        plsc.subcore_barrier()
        pltpu.sync_copy(deg_sh.at[pl.ds(soff, STRIPE)],
                        out_hbm.at[pl.ds(soff, STRIPE)])


def _run_sc_deg(dst_pad, zeros1d):
    f = functools.partial(
        pl.kernel,
        out_type=jax.ShapeDtypeStruct((NP,), jnp.float32),
        mesh=_SC_MESH,
        scratch_types=[
            pltpu.VMEM((CHUNK,), jnp.int32),
            pltpu.VMEM((CHUNK,), jnp.float32),
            pltpu.VMEM_SHARED((NP,), jnp.float32),
        ],
    )(_sc_deg_body)
    return f(dst_pad, zeros1d)


def _sc_agg_body(y_hbm, src_hbm, dst_hbm, zeros_hbm, out_hbm,
                 srcv, dstv, rows_v, acc_sh, sem):
    cid = lax.axis_index("c")
    sid = lax.axis_index("s")
    wid = cid * jnp.int32(16) + sid
    soff = sid * jnp.int32(STRIPE)

    pltpu.sync_copy(zeros_hbm.at[pl.ds(soff, STRIPE)],
                    acc_sh.at[pl.ds(soff, STRIPE)])
    plsc.subcore_barrier()

    def body(k, carry):
        base = wid * jnp.int32(EPT) + k * jnp.int32(CHUNK)
        pltpu.sync_copy(src_hbm.at[pl.ds(base, CHUNK)], srcv)
        pltpu.sync_copy(dst_hbm.at[pl.ds(base, CHUNK)], dstv)
        pltpu.async_copy(y_hbm.at[srcv], rows_v, sem).wait()
        pltpu.sync_copy(rows_v, acc_sh.at[dstv], add=True)
        return carry

    lax.fori_loop(jnp.int32(0), jnp.int32(NCHUNK), body, None)

    plsc.subcore_barrier()
    pltpu.sync_copy(acc_sh.at[pl.ds(soff, STRIPE)],
                    out_hbm.at[cid, pl.ds(soff, STRIPE)])


def _run_sc_agg(y, src_pad, dst_pad, zeros2d):
    f = functools.partial(
        pl.kernel,
        out_type=jax.ShapeDtypeStruct((2, NP, DX), jnp.float32),
        mesh=_SC_MESH,
        scratch_types=[
            pltpu.VMEM((CHUNK,), jnp.int32),
            pltpu.VMEM((CHUNK,), jnp.int32),
            pltpu.VMEM((CHUNK, DX), jnp.float32),
            pltpu.VMEM_SHARED((NP, DX), jnp.float32),
            pltpu.SemaphoreType.DMA,
        ],
    )(_sc_agg_body)
    return f(y, src_pad, dst_pad, zeros2d)


BR = 2000


def _k4_body(xn_ref, gx_ref, deg_ref, wea_ref, web_ref, be1_ref, wg1_ref,
             z_ref, dinv_ref, xw1_ref, y1_ref):
    z = jnp.dot(xn_ref[...], wea_ref[...], preferred_element_type=jnp.float32)
    z = z + jnp.dot(gx_ref[...], web_ref[...],
                    preferred_element_type=jnp.float32)
    z = jnp.maximum(z + be1_ref[...], 0.0)
    z_ref[...] = z

    deg = deg_ref[...] + 1.0
    dinv = lax.rsqrt(jnp.maximum(deg, 1.0))
    dinv_ref[...] = dinv

    xw1 = jnp.dot(z + gx_ref[...], wg1_ref[...],
                  preferred_element_type=jnp.float32)
    xw1_ref[...] = xw1
    y1_ref[...] = dinv * xw1


def _run_k4(xn, gx, deg2d, wea, web, be1, wg1):
    return pl.pallas_call(
        _k4_body,
        grid=(N // BR,),
        in_specs=[
            pl.BlockSpec((BR, DX), lambda r: (r, jnp.int32(0))),
            pl.BlockSpec((BR, DX), lambda r: (r, jnp.int32(0))),
            pl.BlockSpec((BR, 1), lambda r: (r, jnp.int32(0))),
            pl.BlockSpec((DX, DX), lambda r: (jnp.int32(0), jnp.int32(0))),
            pl.BlockSpec((DX, DX), lambda r: (jnp.int32(0), jnp.int32(0))),
            pl.BlockSpec((1, DX), lambda r: (jnp.int32(0), jnp.int32(0))),
            pl.BlockSpec((DX, DX), lambda r: (jnp.int32(0), jnp.int32(0))),
        ],
        out_specs=[
            pl.BlockSpec((BR, DX), lambda r: (r, jnp.int32(0))),
            pl.BlockSpec((BR, 1), lambda r: (r, jnp.int32(0))),
            pl.BlockSpec((BR, DX), lambda r: (r, jnp.int32(0))),
            pl.BlockSpec((BR, DX), lambda r: (r, jnp.int32(0))),
        ],
        out_shape=[
            jax.ShapeDtypeStruct((N, DX), jnp.float32),
            jax.ShapeDtypeStruct((N, 1), jnp.float32),
            jax.ShapeDtypeStruct((N, DX), jnp.float32),
            jax.ShapeDtypeStruct((N, DX), jnp.float32),
        ],
    )(xn, gx, deg2d, wea, web, be1, wg1)


def _k5_body(agg_ref, xw1_ref, dinv_ref, bg1_ref, wg2_ref,
             z1_ref, xw2_ref, y2_ref):
    agg = agg_ref[0] + agg_ref[1]
    dinv = dinv_ref[...]
    z1 = jnp.maximum(dinv * agg + dinv * dinv * xw1_ref[...] + bg1_ref[...],
                     0.0)
    z1_ref[...] = z1
    xw2 = jnp.dot(z1, wg2_ref[...], preferred_element_type=jnp.float32)
    xw2_ref[...] = xw2
    y2_ref[...] = dinv * xw2


def _run_k5(agg1, xw1, dinv, bg1, wg2):
    return pl.pallas_call(
        _k5_body,
        grid=(N // BR,),
        in_specs=[
            pl.BlockSpec((2, BR, DX), lambda r: (jnp.int32(0), r, jnp.int32(0))),
            pl.BlockSpec((BR, DX), lambda r: (r, jnp.int32(0))),
            pl.BlockSpec((BR, 1), lambda r: (r, jnp.int32(0))),
            pl.BlockSpec((1, DX), lambda r: (jnp.int32(0), jnp.int32(0))),
            pl.BlockSpec((DX, DX), lambda r: (jnp.int32(0), jnp.int32(0))),
        ],
        out_specs=[
            pl.BlockSpec((BR, DX), lambda r: (r, jnp.int32(0))),
            pl.BlockSpec((BR, DX), lambda r: (r, jnp.int32(0))),
            pl.BlockSpec((BR, DX), lambda r: (r, jnp.int32(0))),
        ],
        out_shape=[
            jax.ShapeDtypeStruct((N, DX), jnp.float32),
            jax.ShapeDtypeStruct((N, DX), jnp.float32),
            jax.ShapeDtypeStruct((N, DX), jnp.float32),
        ],
    )(agg1, xw1, dinv, bg1, wg2)


def _k6_body(agg_ref, xw2_ref, dinv_ref, z_ref, z1_ref, bg2_ref,
             wa_ref, wb_ref, wc_ref, be2_ref, we3_ref, be3_ref,
             wo_ref, bo_ref, out_ref):
    agg = agg_ref[0] + agg_ref[1]
    dinv = dinv_ref[...]
    z2 = jnp.maximum(dinv * agg + dinv * dinv * xw2_ref[...] + bg2_ref[...],
                     0.0)
    t = jnp.dot(z_ref[...], wa_ref[...], preferred_element_type=jnp.float32)
    t = t + jnp.dot(z1_ref[...], wb_ref[...],
                    preferred_element_type=jnp.float32)
    t = t + jnp.dot(z2, wc_ref[...], preferred_element_type=jnp.float32)
    t = jnp.maximum(t + be2_ref[...], 0.0)
    t = jnp.maximum(jnp.dot(t, we3_ref[...],
                            preferred_element_type=jnp.float32) + be3_ref[...],
                    0.0)
    out_ref[...] = jnp.dot(t, wo_ref[...],
                           preferred_element_type=jnp.float32) + bo_ref[...]


def _run_k6(agg2, xw2, dinv, z, z1, bg2, wa, wb, wc, be2, we3, be3, wo, bo):
    nl = wo.shape[1]
    return pl.pallas_call(
        _k6_body,
        grid=(N // BR,),
        in_specs=[
            pl.BlockSpec((2, BR, DX), lambda r: (jnp.int32(0), r, jnp.int32(0))),
            pl.BlockSpec((BR, DX), lambda r: (r, jnp.int32(0))),
            pl.BlockSpec((BR, 1), lambda r: (r, jnp.int32(0))),
            pl.BlockSpec((BR, DX), lambda r: (r, jnp.int32(0))),
            pl.BlockSpec((BR, DX), lambda r: (r, jnp.int32(0))),
            pl.BlockSpec((1, DX), lambda r: (jnp.int32(0), jnp.int32(0))),
            pl.BlockSpec((DX, DX), lambda r: (jnp.int32(0), jnp.int32(0))),
            pl.BlockSpec((DX, DX), lambda r: (jnp.int32(0), jnp.int32(0))),
            pl.BlockSpec((DX, DX), lambda r: (jnp.int32(0), jnp.int32(0))),
            pl.BlockSpec((1, DX), lambda r: (jnp.int32(0), jnp.int32(0))),
            pl.BlockSpec((DX, DX), lambda r: (jnp.int32(0), jnp.int32(0))),
            pl.BlockSpec((1, DX), lambda r: (jnp.int32(0), jnp.int32(0))),
            pl.BlockSpec((DX, nl), lambda r: (jnp.int32(0), jnp.int32(0))),
            pl.BlockSpec((1, nl), lambda r: (jnp.int32(0), jnp.int32(0))),
        ],
        out_specs=pl.BlockSpec((BR, nl), lambda r: (r, jnp.int32(0))),
        out_shape=jax.ShapeDtypeStruct((N, nl), jnp.float32),
    )(agg2, xw2, dinv, z, z1, bg2, wa, wb, wc, be2, we3, be3, wo, bo)


def kernel(x, gx, edge_index, W_e1, b_e1, W_dr, b_dr, W_g1, b_g1, W_g2, b_g2,
           W_e2, b_e2, W_e3, b_e3, W_j1, b_j1, W_j2, b_j2, W_out, b_out):
    f32 = jnp.float32
    (x, gx, W_e1, b_e1, W_g1, b_g1, W_g2, b_g2, W_e2, b_e2, W_e3, b_e3,
     W_j1, b_j1, W_j2, b_j2, W_out, b_out) = (
        a.astype(f32) for a in
        (x, gx, W_e1, b_e1, W_g1, b_g1, W_g2, b_g2, W_e2, b_e2, W_e3, b_e3,
         W_j1, b_j1, W_j2, b_j2, W_out, b_out))

    w_j1a, w_j1b = W_j1[:DX], W_j1[DX:]
    b_j1r = b_j1.reshape(1, HJ)
    w_j2p = jnp.pad(W_j2, ((0, 0), (0, NP - N)))
    b_j2p = jnp.pad(b_j2, (0, NP - N)).reshape(1, NP)
    x_pad = jnp.pad(x, ((0, NP - N), (0, 0)))

    src = edge_index[0].astype(jnp.int32)
    dst = edge_index[1].astype(jnp.int32)
    src_pad = jnp.pad(src, (0, E3 - E))
    dst_pad = jnp.pad(dst, (0, E3 - E), constant_values=N)
    zeros1d = jnp.zeros((NP,), f32)
    zeros2d = jnp.zeros((NP, DX), f32)

    w_e1a, w_e1b = W_e1[:DX], W_e1[DX:]
    w_e2a, w_e2b, w_e2c = W_e2[:DX], W_e2[DX:2 * DX], W_e2[2 * DX:]

    deg1d = _run_sc_deg(dst_pad, zeros1d)
    deg2d = deg1d.reshape(NP, 1)

    zz, dinv = _run_k2(x, gx, w_j1a, w_j1b, b_j1r, w_j2p, b_j2p)
    dinv_row = jnp.concatenate(
        [dinv.reshape(1, N), jnp.ones((1, NP - N), f32)], axis=1)
    xn = _run_k3(zz, dinv, dinv_row, x_pad)

    z, gdinv, xw1, y1 = _run_k4(xn, gx, deg2d, w_e1a, w_e1b,
                                b_e1.reshape(1, DX), W_g1)
    agg1 = _run_sc_agg(y1, src_pad, dst_pad, zeros2d)
    z1, xw2, y2 = _run_k5(agg1, xw1, gdinv, b_g1.reshape(1, DX), W_g2)
    agg2 = _run_sc_agg(y2, src_pad, dst_pad, zeros2d)
    out = _run_k6(agg2, xw2, gdinv, z, z1, b_g2.reshape(1, DX),
                  w_e2a, w_e2b, w_e2c, b_e2.reshape(1, DX),
                  W_e3, b_e3.reshape(1, DX), W_out, b_out.reshape(1, -1))
    return out.astype(jnp.float64)


# speedup vs baseline: 1.1339x; 1.0511x over previous
import functools

import jax
import jax.numpy as jnp
from jax import lax
from jax.experimental import pallas as pl
from jax.experimental.pallas import tpu as pltpu
from jax.experimental.pallas import tpu_sc as plsc

N = 10000
NP = 10240
DX = 128
HJ = 2048
ALPHA = 0.05
C1 = ALPHA * (1.0 - ALPHA)
TOPK = 128

E = 320000
CHUNK = 128
NTILES = 32
NCHUNK = 79
EPT = NCHUNK * CHUNK
E3 = NTILES * EPT
STRIPE = NP // 16

_SC_MESH = plsc.VectorSubcoreMesh(core_axis_name="c", subcore_axis_name="s")


BI2 = 1000
BJ2 = 640


def _k2_body(x_ref, gx_ref, wa_ref, wb_ref, b1_ref, w2_ref, b2_ref,
             zz_ref, dinv_ref, h_s):
    j = pl.program_id(1)
    nj = pl.num_programs(1)

    @pl.when(j == 0)
    def _():
        h = jnp.dot(x_ref[...], wa_ref[...], preferred_element_type=jnp.float32)
        h = h + jnp.dot(gx_ref[...], wb_ref[...],
                        preferred_element_type=jnp.float32)
        h_s[...] = jnp.maximum(h + b1_ref[...], 0.0)

    zz = jnp.dot(h_s[...], w2_ref[...], preferred_element_type=jnp.float32)
    zz = jnp.maximum(zz + b2_ref[...], 0.0)
    zz_ref[...] = zz
    rs = jnp.sum(zz, axis=1, keepdims=True)

    @pl.when(j == 0)
    def _():
        dinv_ref[...] = rs + 1.0

    @pl.when(j > 0)
    def _():
        dinv_ref[...] = dinv_ref[...] + rs

    @pl.when(j == nj - 1)
    def _():
        dinv_ref[...] = lax.rsqrt(dinv_ref[...])


def _run_k2(x, gx, wa, wb, b1, w2p, b2p):
    return pl.pallas_call(
        _k2_body,
        grid=(N // BI2, NP // BJ2),
        in_specs=[
            pl.BlockSpec((BI2, DX), lambda i, j: (i, jnp.int32(0))),
            pl.BlockSpec((BI2, DX), lambda i, j: (i, jnp.int32(0))),
            pl.BlockSpec((DX, HJ), lambda i, j: (jnp.int32(0), jnp.int32(0))),
            pl.BlockSpec((DX, HJ), lambda i, j: (jnp.int32(0), jnp.int32(0))),
            pl.BlockSpec((1, HJ), lambda i, j: (jnp.int32(0), jnp.int32(0))),
            pl.BlockSpec((HJ, BJ2), lambda i, j: (jnp.int32(0), j)),
            pl.BlockSpec((1, BJ2), lambda i, j: (jnp.int32(0), j)),
        ],
        out_specs=[
            pl.BlockSpec((BI2, BJ2), lambda i, j: (i, j)),
            pl.BlockSpec((BI2, 1), lambda i, j: (i, jnp.int32(0))),
        ],
        out_shape=[
            jax.ShapeDtypeStruct((N, NP), jnp.float32),
            jax.ShapeDtypeStruct((N, 1), jnp.float32),
        ],
        scratch_shapes=[pltpu.VMEM((BI2, HJ), jnp.float32)],
    )(x, gx, wa, wb, b1, w2p, b2p)


BC3 = 128


def _k3_body(zz_ref, dc_ref, dr_ref, xp_ref, xn_ref, s_s):
    j = pl.program_id(0)

    dc = dc_ref[...]
    s = (C1 * dc) * zz_ref[...] * dr_ref[...]
    rows = lax.broadcasted_iota(jnp.int32, (N, BC3), 0)
    cols = (j * jnp.int32(BC3)) + lax.broadcasted_iota(jnp.int32, (N, BC3), 1)
    s = jnp.where(rows == cols, s + (C1 * dc * dc + ALPHA), s)
    s_s[...] = s

    lo0 = jnp.zeros((1, BC3), jnp.int32)
    hi0 = jnp.full((1, BC3), 0x7F7FFFFF, jnp.int32)

    def step(_, carry):
        lo, hi = carry
        span = hi - lo
        q = (span + 3) // 4
        m1 = lo + q
        m2 = jnp.minimum(m1 + q, hi)
        m3 = jnp.minimum(m2 + q, hi)
        t1 = lax.bitcast_convert_type(m1, jnp.float32)
        t2 = lax.bitcast_convert_type(m2, jnp.float32)
        t3 = lax.bitcast_convert_type(m3, jnp.float32)
        sv = s_s[...]
        c1 = jnp.sum((sv >= t1).astype(jnp.float32), axis=0, keepdims=True)
        c2 = jnp.sum((sv >= t2).astype(jnp.float32), axis=0, keepdims=True)
        c3 = jnp.sum((sv >= t3).astype(jnp.float32), axis=0, keepdims=True)
        p1 = c1 >= float(TOPK)
        p2 = c2 >= float(TOPK)
        p3 = c3 >= float(TOPK)
        lo = jnp.where(p3, m3, jnp.where(p2, m2, jnp.where(p1, m1, lo)))
        hi = jnp.where(p3, hi,
                       jnp.where(p2, m3 - 1, jnp.where(p1, m2 - 1, m1 - 1)))
        return lo, hi

    lo, _ = lax.fori_loop(jnp.int32(0), jnp.int32(16), step, (lo0, hi0))
    kth = lax.bitcast_convert_type(lo, jnp.float32)

    sm = jnp.where(s_s[...] >= kth, s_s[...], 0.0)
    col = jnp.sum(sm, axis=0, keepdims=True)
    sm = sm / jnp.where(col == 0.0, 1.0, col)
    contrib = jnp.dot(sm, xp_ref[...], preferred_element_type=jnp.float32)

    @pl.when(j == 0)
    def _():
        xn_ref[...] = contrib

    @pl.when(j > 0)
    def _():
        xn_ref[...] = xn_ref[...] + contrib


def _run_k3(zz, dinv, dinv_row, x_pad):
    return pl.pallas_call(
        _k3_body,
        grid=(NP // BC3,),
        in_specs=[
            pl.BlockSpec((N, BC3), lambda j: (jnp.int32(0), j)),
            pl.BlockSpec((N, 1), lambda j: (jnp.int32(0), jnp.int32(0))),
            pl.BlockSpec((1, BC3), lambda j: (jnp.int32(0), j)),
            pl.BlockSpec((BC3, DX), lambda j: (j, jnp.int32(0))),
        ],
        out_specs=pl.BlockSpec((N, DX), lambda j: (jnp.int32(0), jnp.int32(0))),
        out_shape=jax.ShapeDtypeStruct((N, DX), jnp.float32),
        scratch_shapes=[pltpu.VMEM((N, BC3), jnp.float32)],
    )(zz, dinv, dinv_row, x_pad)


def _sc_deg_body(dst_hbm, zeros_hbm, out_hbm, dstv, onesv, deg_sh):
    cid = lax.axis_index("c")
    sid = lax.axis_index("s")
    soff = sid * jnp.int32(STRIPE)

    @pl.when(cid == 0)
    def _():
        pltpu.sync_copy(zeros_hbm.at[pl.ds(soff, STRIPE)],
                        deg_sh.at[pl.ds(soff, STRIPE)])
        for jj in range(CHUNK // 16):
            onesv[pl.ds(jj * 16, 16)] = jnp.ones((16,), jnp.float32)
        plsc.subcore_barrier()

        for seg in range(2):
            def body(k, carry, seg=seg):
                base = ((sid + jnp.int32(16 * seg)) * jnp.int32(EPT)
                        + k * jnp.int32(CHUNK))
                pltpu.sync_copy(dst_hbm.at[pl.ds(base, CHUNK)], dstv)
                pltpu.sync_copy(onesv, deg_sh.at[dstv], add=True)
                return carry

            lax.fori_loop(jnp.int32(0), jnp.int32(NCHUNK), body, None)

        plsc.subcore_barrier()
        pltpu.sync_copy(deg_sh.at[pl.ds(soff, STRIPE)],
                        out_hbm.at[pl.ds(soff, STRIPE)])


def _run_sc_deg(dst_pad, zeros1d):
    f = functools.partial(
        pl.kernel,
        out_type=jax.ShapeDtypeStruct((NP,), jnp.float32),
        mesh=_SC_MESH,
        scratch_types=[
            pltpu.VMEM((CHUNK,), jnp.int32),
            pltpu.VMEM((CHUNK,), jnp.float32),
            pltpu.VMEM_SHARED((NP,), jnp.float32),
        ],
    )(_sc_deg_body)
    return f(dst_pad, zeros1d)


def _sc_agg_body(y_hbm, src_hbm, dst_hbm, zeros_hbm, out_hbm,
                 srcv, dstv, rows_v, acc_sh, sem):
    cid = lax.axis_index("c")
    sid = lax.axis_index("s")
    wid = cid * jnp.int32(16) + sid
    soff = sid * jnp.int32(STRIPE)

    pltpu.sync_copy(zeros_hbm.at[pl.ds(soff, STRIPE)],
                    acc_sh.at[pl.ds(soff, STRIPE)])
    plsc.subcore_barrier()

    def body(k, carry):
        base = wid * jnp.int32(EPT) + k * jnp.int32(CHUNK)
        pltpu.sync_copy(src_hbm.at[pl.ds(base, CHUNK)], srcv)
        pltpu.sync_copy(dst_hbm.at[pl.ds(base, CHUNK)], dstv)
        pltpu.async_copy(y_hbm.at[srcv], rows_v, sem).wait()
        pltpu.sync_copy(rows_v, acc_sh.at[dstv], add=True)
        return carry

    lax.fori_loop(jnp.int32(0), jnp.int32(NCHUNK), body, None)

    plsc.subcore_barrier()
    pltpu.sync_copy(acc_sh.at[pl.ds(soff, STRIPE)],
                    out_hbm.at[cid, pl.ds(soff, STRIPE)])


def _run_sc_agg(y, src_pad, dst_pad, zeros2d):
    f = functools.partial(
        pl.kernel,
        out_type=jax.ShapeDtypeStruct((2, NP, DX), jnp.float32),
        mesh=_SC_MESH,
        scratch_types=[
            pltpu.VMEM((CHUNK,), jnp.int32),
            pltpu.VMEM((CHUNK,), jnp.int32),
            pltpu.VMEM((CHUNK, DX), jnp.float32),
            pltpu.VMEM_SHARED((NP, DX), jnp.float32),
            pltpu.SemaphoreType.DMA,
        ],
    )(_sc_agg_body)
    return f(y, src_pad, dst_pad, zeros2d)


BR = 2000


def _k4_body(xn_ref, gx_ref, deg_ref, wea_ref, web_ref, be1_ref, wg1_ref,
             z_ref, dinv_ref, xw1_ref, y1_ref):
    z = jnp.dot(xn_ref[...], wea_ref[...], preferred_element_type=jnp.float32)
    z = z + jnp.dot(gx_ref[...], web_ref[...],
                    preferred_element_type=jnp.float32)
    z = jnp.maximum(z + be1_ref[...], 0.0)
    z_ref[...] = z

    deg = deg_ref[...] + 1.0
    dinv = lax.rsqrt(jnp.maximum(deg, 1.0))
    dinv_ref[...] = dinv

    xw1 = jnp.dot(z + gx_ref[...], wg1_ref[...],
                  preferred_element_type=jnp.float32)
    xw1_ref[...] = xw1
    y1_ref[...] = dinv * xw1


def _run_k4(xn, gx, deg2d, wea, web, be1, wg1):
    return pl.pallas_call(
        _k4_body,
        grid=(N // BR,),
        in_specs=[
            pl.BlockSpec((BR, DX), lambda r: (r, jnp.int32(0))),
            pl.BlockSpec((BR, DX), lambda r: (r, jnp.int32(0))),
            pl.BlockSpec((BR, 1), lambda r: (r, jnp.int32(0))),
            pl.BlockSpec((DX, DX), lambda r: (jnp.int32(0), jnp.int32(0))),
            pl.BlockSpec((DX, DX), lambda r: (jnp.int32(0), jnp.int32(0))),
            pl.BlockSpec((1, DX), lambda r: (jnp.int32(0), jnp.int32(0))),
            pl.BlockSpec((DX, DX), lambda r: (jnp.int32(0), jnp.int32(0))),
        ],
        out_specs=[
            pl.BlockSpec((BR, DX), lambda r: (r, jnp.int32(0))),
            pl.BlockSpec((BR, 1), lambda r: (r, jnp.int32(0))),
            pl.BlockSpec((BR, DX), lambda r: (r, jnp.int32(0))),
            pl.BlockSpec((BR, DX), lambda r: (r, jnp.int32(0))),
        ],
        out_shape=[
            jax.ShapeDtypeStruct((N, DX), jnp.float32),
            jax.ShapeDtypeStruct((N, 1), jnp.float32),
            jax.ShapeDtypeStruct((N, DX), jnp.float32),
            jax.ShapeDtypeStruct((N, DX), jnp.float32),
        ],
    )(xn, gx, deg2d, wea, web, be1, wg1)


def _k5_body(agg_ref, xw1_ref, dinv_ref, bg1_ref, wg2_ref,
             z1_ref, xw2_ref, y2_ref):
    agg = agg_ref[0] + agg_ref[1]
    dinv = dinv_ref[...]
    z1 = jnp.maximum(dinv * agg + dinv * dinv * xw1_ref[...] + bg1_ref[...],
                     0.0)
    z1_ref[...] = z1
    xw2 = jnp.dot(z1, wg2_ref[...], preferred_element_type=jnp.float32)
    xw2_ref[...] = xw2
    y2_ref[...] = dinv * xw2


def _run_k5(agg1, xw1, dinv, bg1, wg2):
    return pl.pallas_call(
        _k5_body,
        grid=(N // BR,),
        in_specs=[
            pl.BlockSpec((2, BR, DX), lambda r: (jnp.int32(0), r, jnp.int32(0))),
            pl.BlockSpec((BR, DX), lambda r: (r, jnp.int32(0))),
            pl.BlockSpec((BR, 1), lambda r: (r, jnp.int32(0))),
            pl.BlockSpec((1, DX), lambda r: (jnp.int32(0), jnp.int32(0))),
            pl.BlockSpec((DX, DX), lambda r: (jnp.int32(0), jnp.int32(0))),
        ],
        out_specs=[
            pl.BlockSpec((BR, DX), lambda r: (r, jnp.int32(0))),
            pl.BlockSpec((BR, DX), lambda r: (r, jnp.int32(0))),
            pl.BlockSpec((BR, DX), lambda r: (r, jnp.int32(0))),
        ],
        out_shape=[
            jax.ShapeDtypeStruct((N, DX), jnp.float32),
            jax.ShapeDtypeStruct((N, DX), jnp.float32),
            jax.ShapeDtypeStruct((N, DX), jnp.float32),
        ],
    )(agg1, xw1, dinv, bg1, wg2)


def _k6_body(agg_ref, xw2_ref, dinv_ref, z_ref, z1_ref, bg2_ref,
             wa_ref, wb_ref, wc_ref, be2_ref, we3_ref, be3_ref,
             wo_ref, bo_ref, out_ref):
    agg = agg_ref[0] + agg_ref[1]
    dinv = dinv_ref[...]
    z2 = jnp.maximum(dinv * agg + dinv * dinv * xw2_ref[...] + bg2_ref[...],
                     0.0)
    t = jnp.dot(z_ref[...], wa_ref[...], preferred_element_type=jnp.float32)
    t = t + jnp.dot(z1_ref[...], wb_ref[...],
                    preferred_element_type=jnp.float32)
    t = t + jnp.dot(z2, wc_ref[...], preferred_element_type=jnp.float32)
    t = jnp.maximum(t + be2_ref[...], 0.0)
    t = jnp.maximum(jnp.dot(t, we3_ref[...],
                            preferred_element_type=jnp.float32) + be3_ref[...],
                    0.0)
    out_ref[...] = jnp.dot(t, wo_ref[...],
                           preferred_element_type=jnp.float32) + bo_ref[...]


def _run_k6(agg2, xw2, dinv, z, z1, bg2, wa, wb, wc, be2, we3, be3, wo, bo):
    nl = wo.shape[1]
    return pl.pallas_call(
        _k6_body,
        grid=(N // BR,),
        in_specs=[
            pl.BlockSpec((2, BR, DX), lambda r: (jnp.int32(0), r, jnp.int32(0))),
            pl.BlockSpec((BR, DX), lambda r: (r, jnp.int32(0))),
            pl.BlockSpec((BR, 1), lambda r: (r, jnp.int32(0))),
            pl.BlockSpec((BR, DX), lambda r: (r, jnp.int32(0))),
            pl.BlockSpec((BR, DX), lambda r: (r, jnp.int32(0))),
            pl.BlockSpec((1, DX), lambda r: (jnp.int32(0), jnp.int32(0))),
            pl.BlockSpec((DX, DX), lambda r: (jnp.int32(0), jnp.int32(0))),
            pl.BlockSpec((DX, DX), lambda r: (jnp.int32(0), jnp.int32(0))),
            pl.BlockSpec((DX, DX), lambda r: (jnp.int32(0), jnp.int32(0))),
            pl.BlockSpec((1, DX), lambda r: (jnp.int32(0), jnp.int32(0))),
            pl.BlockSpec((DX, DX), lambda r: (jnp.int32(0), jnp.int32(0))),
            pl.BlockSpec((1, DX), lambda r: (jnp.int32(0), jnp.int32(0))),
            pl.BlockSpec((DX, nl), lambda r: (jnp.int32(0), jnp.int32(0))),
            pl.BlockSpec((1, nl), lambda r: (jnp.int32(0), jnp.int32(0))),
        ],
        out_specs=pl.BlockSpec((BR, nl), lambda r: (r, jnp.int32(0))),
        out_shape=jax.ShapeDtypeStruct((N, nl), jnp.float32),
    )(agg2, xw2, dinv, z, z1, bg2, wa, wb, wc, be2, we3, be3, wo, bo)


def kernel(x, gx, edge_index, W_e1, b_e1, W_dr, b_dr, W_g1, b_g1, W_g2, b_g2,
           W_e2, b_e2, W_e3, b_e3, W_j1, b_j1, W_j2, b_j2, W_out, b_out):
    f32 = jnp.float32
    (x, gx, W_e1, b_e1, W_g1, b_g1, W_g2, b_g2, W_e2, b_e2, W_e3, b_e3,
     W_j1, b_j1, W_j2, b_j2, W_out, b_out) = (
        a.astype(f32) for a in
        (x, gx, W_e1, b_e1, W_g1, b_g1, W_g2, b_g2, W_e2, b_e2, W_e3, b_e3,
         W_j1, b_j1, W_j2, b_j2, W_out, b_out))

    w_j1a, w_j1b = W_j1[:DX], W_j1[DX:]
    b_j1r = b_j1.reshape(1, HJ)
    w_j2p = jnp.pad(W_j2, ((0, 0), (0, NP - N)))
    b_j2p = jnp.pad(b_j2, (0, NP - N)).reshape(1, NP)
    x_pad = jnp.pad(x, ((0, NP - N), (0, 0)))

    src = edge_index[0].astype(jnp.int32)
    dst = edge_index[1].astype(jnp.int32)
    src_pad = jnp.pad(src, (0, E3 - E))
    dst_pad = jnp.pad(dst, (0, E3 - E), constant_values=N)
    zeros1d = jnp.zeros((NP,), f32)
    zeros2d = jnp.zeros((NP, DX), f32)

    w_e1a, w_e1b = W_e1[:DX], W_e1[DX:]
    w_e2a, w_e2b, w_e2c = W_e2[:DX], W_e2[DX:2 * DX], W_e2[2 * DX:]

    deg1d = _run_sc_deg(dst_pad, zeros1d)
    deg2d = deg1d.reshape(NP, 1)

    zz, dinv = _run_k2(x, gx, w_j1a, w_j1b, b_j1r, w_j2p, b_j2p)
    dinv_row = jnp.concatenate(
        [dinv.reshape(1, N), jnp.ones((1, NP - N), f32)], axis=1)
    xn = _run_k3(zz, dinv, dinv_row, x_pad)

    z, gdinv, xw1, y1 = _run_k4(xn, gx, deg2d, w_e1a, w_e1b,
                                b_e1.reshape(1, DX), W_g1)
    agg1 = _run_sc_agg(y1, src_pad, dst_pad, zeros2d)
    z1, xw2, y2 = _run_k5(agg1, xw1, gdinv, b_g1.reshape(1, DX), W_g2)
    agg2 = _run_sc_agg(y2, src_pad, dst_pad, zeros2d)
    out = _run_k6(agg2, xw2, gdinv, z, z1, b_g2.reshape(1, DX),
                  w_e2a, w_e2b, w_e2c, b_e2.reshape(1, DX),
                  W_e3, b_e3.reshape(1, DX), W_out, b_out.reshape(1, -1))
    return out.astype(jnp.float64)
